# baseline (speedup 1.0000x reference)
import functools

import jax
import jax.numpy as jnp
from jax import lax
from jax.experimental import pallas as pl
from jax.experimental.pallas import tpu as pltpu
from jax.experimental.pallas import tpu_sc as plsc

_MAX_DIS = 512
_DIM = 128
_N = 819200

_NC = 2
_NS = 16
_NW = _NC * _NS
_B_PER_W = _N // _NW
_CHUNK = 128
_NSTEPS = _B_PER_W // _CHUNK
_LANES = 16
_VPR = _DIM // _LANES


def _body(d_hbm, table_hbm, out_hbm, table_sp, table_l, idx_all,
          rows0, rows1, si, so0, so1):
    rows = (rows0, rows1)
    so = (so0, so1)

    sid = lax.axis_index("s")
    wid = sid * _NC + lax.axis_index("c")
    base = wid * _B_PER_W

    @pl.when(sid == 0)
    def _():
        pltpu.sync_copy(table_hbm, table_sp)

    idx_cp = pltpu.make_async_copy(
        d_hbm.at[pl.ds(base, _B_PER_W)], idx_all, si)
    idx_cp.start()

    plsc.subcore_barrier()

    pltpu.sync_copy(table_sp, table_l)
    idx_cp.wait()

    def compute_chunk(b, g):
        goff = g * _CHUNK

        def group(j, carry):
            v = idx_all[pl.ds(goff + j * _LANES, _LANES)]
            v = jnp.where(v > 1000, _MAX_DIS + 1, jnp.minimum(v, _MAX_DIS))
            for k in range(_LANES):
                r = v[k]
                dr = j * _LANES + k
                for c in range(_VPR):
                    rows[b][dr, pl.ds(c * _LANES, _LANES)] = (
                        table_l[r, pl.ds(c * _LANES, _LANES)])
            return carry

        lax.fori_loop(0, _CHUNK // _LANES, group, 0)

    def start_out(b, g):
        off = base + g * _CHUNK
        pltpu.make_async_copy(
            rows[b], out_hbm.at[pl.ds(off, _CHUNK)], so[b]).start()

    def wait_out(b, g):
        off = base + g * _CHUNK
        pltpu.make_async_copy(
            rows[b], out_hbm.at[pl.ds(off, _CHUNK)], so[b]).wait()

    compute_chunk(0, 0)
    start_out(0, 0)
    compute_chunk(1, 1)
    wait_out(0, 0)
    start_out(1, 1)

    def step(i, carry):
        for b in range(2):
            g = 2 * i + b
            compute_chunk(b, g)
            wait_out(1 - b, g - 1)
            start_out(b, g)
        return carry

    lax.fori_loop(1, _NSTEPS // 2, step, 0)

    wait_out(1, _NSTEPS - 1)


_mesh = plsc.VectorSubcoreMesh(core_axis_name="c", subcore_axis_name="s")

_gather = functools.partial(
    pl.kernel,
    out_type=jax.ShapeDtypeStruct((_N, _DIM), jnp.float32),
    mesh=_mesh,
    scratch_types=[
        pltpu.VMEM_SHARED((_MAX_DIS + 2, _DIM), jnp.float32),
        pltpu.VMEM((_MAX_DIS + 2, _DIM), jnp.float32),
        pltpu.VMEM((_B_PER_W,), jnp.int32),
        pltpu.VMEM((_CHUNK, _DIM), jnp.float32),
        pltpu.VMEM((_CHUNK, _DIM), jnp.float32),
        pltpu.SemaphoreType.DMA,
        pltpu.SemaphoreType.DMA,
        pltpu.SemaphoreType.DMA,
    ],
)(_body)


def kernel(d, embed_d):
    return _gather(d, embed_d)


# speedup vs baseline: 1.9341x; 1.9341x over previous
import functools

import jax
import jax.numpy as jnp
from jax import lax
from jax.experimental import pallas as pl
from jax.experimental.pallas import tpu as pltpu
from jax.experimental.pallas import tpu_sc as plsc

_MAX_DIS = 512
_DIM = 128
_N = 819200

_NC = 2
_NS = 16
_NW = _NC * _NS
_B_PER_W = _N // _NW
_CHUNK = 128
_NSTEPS = _B_PER_W // _CHUNK
_LANES = 16
_VPR = _DIM // _LANES


def _body(d_hbm, table_hbm, out_hbm, table_sp, table_l, idx_all,
          rows0, rows1, si, so0, so1):
    rows = (rows0, rows1)
    so = (so0, so1)

    sid = lax.axis_index("s")
    wid = sid * _NC + lax.axis_index("c")
    base = wid * _B_PER_W

    @pl.when(sid == 0)
    def _():
        pltpu.sync_copy(table_hbm, table_sp)

    idx_cp = pltpu.make_async_copy(
        d_hbm.at[pl.ds(base, _B_PER_W)], idx_all, si)
    idx_cp.start()

    plsc.subcore_barrier()

    pltpu.sync_copy(table_sp, table_l)
    idx_cp.wait()

    def compute_chunk(b, g):
        goff = g * _CHUNK

        @plsc.parallel_loop(0, _CHUNK // _LANES, unroll=2)
        def _(j):
            v = idx_all[pl.ds(goff + j * _LANES, _LANES)]
            v = jnp.where(v > 1000, _MAX_DIS + 1, jnp.minimum(v, _MAX_DIS))
            for k in range(_LANES):
                r = v[k]
                dr = j * _LANES + k
                for c in range(_VPR):
                    rows[b][dr, pl.ds(c * _LANES, _LANES)] = (
                        table_l[r, pl.ds(c * _LANES, _LANES)])

    def start_out(b, g):
        off = base + g * _CHUNK
        pltpu.make_async_copy(
            rows[b], out_hbm.at[pl.ds(off, _CHUNK)], so[b]).start()

    def wait_out(b, g):
        off = base + g * _CHUNK
        pltpu.make_async_copy(
            rows[b], out_hbm.at[pl.ds(off, _CHUNK)], so[b]).wait()

    compute_chunk(0, 0)
    start_out(0, 0)
    compute_chunk(1, 1)
    wait_out(0, 0)
    start_out(1, 1)

    def step(i, carry):
        for b in range(2):
            g = 2 * i + b
            compute_chunk(b, g)
            wait_out(1 - b, g - 1)
            start_out(b, g)
        return carry

    lax.fori_loop(1, _NSTEPS // 2, step, 0)

    wait_out(1, _NSTEPS - 1)


_mesh = plsc.VectorSubcoreMesh(core_axis_name="c", subcore_axis_name="s")

_gather = functools.partial(
    pl.kernel,
    out_type=jax.ShapeDtypeStruct((_N, _DIM), jnp.float32),
    mesh=_mesh,
    scratch_types=[
        pltpu.VMEM_SHARED((_MAX_DIS + 2, _DIM), jnp.float32),
        pltpu.VMEM((_MAX_DIS + 2, _DIM), jnp.float32),
        pltpu.VMEM((_B_PER_W,), jnp.int32),
        pltpu.VMEM((_CHUNK, _DIM), jnp.float32),
        pltpu.VMEM((_CHUNK, _DIM), jnp.float32),
        pltpu.SemaphoreType.DMA,
        pltpu.SemaphoreType.DMA,
        pltpu.SemaphoreType.DMA,
    ],
)(_body)


def kernel(d, embed_d):
    return _gather(d, embed_d)


# speedup vs baseline: 4.7693x; 2.4658x over previous
import functools

import jax
import jax.numpy as jnp
from jax import lax
from jax.experimental import pallas as pl
from jax.experimental.pallas import tpu as pltpu
from jax.experimental.pallas import tpu_sc as plsc

_MAX_DIS = 512
_DIM = 128
_N = 819200

_NC = 2
_NS = 16
_NW = _NC * _NS
_B_PER_W = _N // _NW
_CHUNK = 128
_NSTEPS = _B_PER_W // _CHUNK
_PIECE = 512
_CPP = _PIECE // _CHUNK
_NPIECE = _B_PER_W // _PIECE
_LANES = 16
_VPR = _DIM // _LANES


def _body(d_hbm, table_hbm, out_hbm, table_sp, d_sp, table_l, smem_idx,
          rows0, rows1, sd, sp0, sp1, so0, so1):
    rows = (rows0, rows1)
    so = (so0, so1)
    sp = (sp0, sp1)

    sid = lax.axis_index("s")
    wid = sid * _NC + lax.axis_index("c")
    base = wid * _B_PER_W

    @pl.when(sid == 0)
    def _():
        pltpu.sync_copy(table_hbm, table_sp)

    d_cp = pltpu.make_async_copy(
        d_hbm.at[pl.ds(base, _B_PER_W)], d_sp.at[sid], sd)
    d_cp.start()

    plsc.subcore_barrier()

    pltpu.sync_copy(table_sp, table_l)
    d_cp.wait()

    def load_piece(pb, p):
        pltpu.make_async_copy(
            d_sp.at[sid, pl.ds(p * _PIECE, _PIECE)], smem_idx.at[pb],
            sp[pb]).start()

    def wait_piece(pb, p):
        pltpu.make_async_copy(
            d_sp.at[sid, pl.ds(p * _PIECE, _PIECE)], smem_idx.at[pb],
            sp[pb]).wait()

    def compute_chunk(b, pb, loc, g):
        @plsc.parallel_loop(0, _CHUNK, unroll=8)
        def _(jrow):
            r = smem_idx[pb, loc * _CHUNK + jrow]
            r = jnp.where(r > 1000, _MAX_DIS + 1, jnp.minimum(r, _MAX_DIS))
            for c in range(_VPR):
                rows[b][jrow, pl.ds(c * _LANES, _LANES)] = (
                    table_l[r, pl.ds(c * _LANES, _LANES)])

    def start_out(b, g):
        off = base + g * _CHUNK
        pltpu.make_async_copy(
            rows[b], out_hbm.at[pl.ds(off, _CHUNK)], so[b]).start()

    def wait_out(b, g):
        off = base + g * _CHUNK
        pltpu.make_async_copy(
            rows[b], out_hbm.at[pl.ds(off, _CHUNK)], so[b]).wait()

    def do_chunk(pb, loc, g, first):
        b = loc % 2
        compute_chunk(b, pb, loc, g)
        if not first:
            wait_out(1 - b, g - 1)
        start_out(b, g)

    load_piece(0, 0)
    wait_piece(0, 0)
    load_piece(1, 1)
    for loc in range(_CPP):
        do_chunk(0, loc, loc, first=(loc == 0))

    def step(i, carry):
        for q in range(2):
            p0 = 2 * i + 1 + q
            pb = (1 + q) % 2
            wait_piece(pb, p0)
            load_piece(1 - pb, p0 + 1)
            for loc in range(_CPP):
                g = p0 * _CPP + loc
                do_chunk(pb, loc, g, first=False)
        return carry

    lax.fori_loop(0, (_NPIECE - 2) // 2, step, 0)

    wait_piece(1, _NPIECE - 1)
    for loc in range(_CPP):
        g = (_NPIECE - 1) * _CPP + loc
        do_chunk(1, loc, g, first=False)

    wait_out((_NSTEPS - 1) % 2, _NSTEPS - 1)


_mesh = plsc.VectorSubcoreMesh(core_axis_name="c", subcore_axis_name="s")

_gather = functools.partial(
    pl.kernel,
    out_type=jax.ShapeDtypeStruct((_N, _DIM), jnp.float32),
    mesh=_mesh,
    scratch_types=[
        pltpu.VMEM_SHARED((_MAX_DIS + 2, _DIM), jnp.float32),
        pltpu.VMEM_SHARED((_NS, _B_PER_W), jnp.int32),
        pltpu.VMEM((_MAX_DIS + 2, _DIM), jnp.float32),
        pltpu.SMEM((2, _PIECE), jnp.int32),
        pltpu.VMEM((_CHUNK, _DIM), jnp.float32),
        pltpu.VMEM((_CHUNK, _DIM), jnp.float32),
        pltpu.SemaphoreType.DMA,
        pltpu.SemaphoreType.DMA,
        pltpu.SemaphoreType.DMA,
        pltpu.SemaphoreType.DMA,
        pltpu.SemaphoreType.DMA,
    ],
)(_body)


def kernel(d, embed_d):
    return _gather(d, embed_d)


# speedup vs baseline: 4.8838x; 1.0240x over previous
import functools

import jax
import jax.numpy as jnp
from jax import lax
from jax.experimental import pallas as pl
from jax.experimental.pallas import tpu as pltpu
from jax.experimental.pallas import tpu_sc as plsc

_MAX_DIS = 512
_DIM = 128
_N = 819200

_NC = 2
_NS = 16
_NW = _NC * _NS
_B_PER_W = _N // _NW
_CHUNK = 128
_NSTEPS = _B_PER_W // _CHUNK
_PIECE = 512
_CPP = _PIECE // _CHUNK
_NPIECE = _B_PER_W // _PIECE
_LANES = 16
_VPR = _DIM // _LANES


def _body(d_hbm, table_hbm, out_hbm, table_sp, d_sp, table_l, smem_idx,
          rows0, rows1, sd, sp0, sp1, so0, so1):
    rows = (rows0, rows1)
    so = (so0, so1)
    sp = (sp0, sp1)

    sid = lax.axis_index("s")
    wid = sid * _NC + lax.axis_index("c")
    base = wid * _B_PER_W

    @pl.when(sid == 0)
    def _():
        pltpu.sync_copy(table_hbm, table_sp)

    d_cp = pltpu.make_async_copy(
        d_hbm.at[pl.ds(base, _B_PER_W)], d_sp.at[sid], sd)
    d_cp.start()

    plsc.subcore_barrier()

    pltpu.sync_copy(table_sp, table_l)
    d_cp.wait()

    def load_piece(pb, p):
        pltpu.make_async_copy(
            d_sp.at[sid, pl.ds(p * _PIECE, _PIECE)], smem_idx.at[pb],
            sp[pb]).start()

    def wait_piece(pb, p):
        pltpu.make_async_copy(
            d_sp.at[sid, pl.ds(p * _PIECE, _PIECE)], smem_idx.at[pb],
            sp[pb]).wait()

    def compute_chunk(b, pb, loc, g):
        @plsc.parallel_loop(0, _CHUNK, unroll=8)
        def _(jrow):
            r = smem_idx[pb, loc * _CHUNK + jrow]
            r = jnp.where(r > 1000, _MAX_DIS + 1, jnp.minimum(r, _MAX_DIS))
            for c in range(_VPR):
                rows[b][jrow, pl.ds(c * _LANES, _LANES)] = (
                    table_l[r, pl.ds(c * _LANES, _LANES)])

    def start_out(b, g):
        off = base + g * _CHUNK
        pltpu.make_async_copy(
            rows[b], out_hbm.at[pl.ds(off, _CHUNK)], so[b]).start()

    def wait_out(b, g):
        off = base + g * _CHUNK
        pltpu.make_async_copy(
            rows[b], out_hbm.at[pl.ds(off, _CHUNK)], so[b]).wait()

    def do_chunk(pb, loc, g, first):
        b = loc % 2
        compute_chunk(b, pb, loc, g)
        start_out(b, g)
        if not first:
            wait_out(1 - b, g - 1)

    load_piece(0, 0)
    wait_piece(0, 0)
    load_piece(1, 1)
    for loc in range(_CPP):
        do_chunk(0, loc, loc, first=(loc == 0))

    def step(i, carry):
        for q in range(2):
            p0 = 2 * i + 1 + q
            pb = (1 + q) % 2
            wait_piece(pb, p0)
            load_piece(1 - pb, p0 + 1)
            for loc in range(_CPP):
                g = p0 * _CPP + loc
                do_chunk(pb, loc, g, first=False)
        return carry

    lax.fori_loop(0, (_NPIECE - 2) // 2, step, 0)

    wait_piece(1, _NPIECE - 1)
    for loc in range(_CPP):
        g = (_NPIECE - 1) * _CPP + loc
        do_chunk(1, loc, g, first=False)

    wait_out((_NSTEPS - 1) % 2, _NSTEPS - 1)


_mesh = plsc.VectorSubcoreMesh(core_axis_name="c", subcore_axis_name="s")

_gather = functools.partial(
    pl.kernel,
    out_type=jax.ShapeDtypeStruct((_N, _DIM), jnp.float32),
    mesh=_mesh,
    scratch_types=[
        pltpu.VMEM_SHARED((_MAX_DIS + 2, _DIM), jnp.float32),
        pltpu.VMEM_SHARED((_NS, _B_PER_W), jnp.int32),
        pltpu.VMEM((_MAX_DIS + 2, _DIM), jnp.float32),
        pltpu.SMEM((2, _PIECE), jnp.int32),
        pltpu.VMEM((_CHUNK, _DIM), jnp.float32),
        pltpu.VMEM((_CHUNK, _DIM), jnp.float32),
        pltpu.SemaphoreType.DMA,
        pltpu.SemaphoreType.DMA,
        pltpu.SemaphoreType.DMA,
        pltpu.SemaphoreType.DMA,
        pltpu.SemaphoreType.DMA,
    ],
)(_body)


def kernel(d, embed_d):
    return _gather(d, embed_d)
